# baseline (speedup 1.0000x reference)
import functools

import jax
import jax.numpy as jnp
from jax import lax
from jax.experimental import pallas as pl
from jax.experimental.pallas import tpu as pltpu
from jax.experimental.pallas import tpu_sc as plsc

_NC = 2
_NS = 16
_NW = _NC * _NS

_CHUNK = 128
_GROUP = 4


@functools.partial(jax.jit, static_argnames=("b_per_w", "d"))
def _gather_call(idx2d, table, *, b_per_w, d):
    chunks_per_w = b_per_w // _CHUNK
    groups = chunks_per_w // _GROUP
    rows_per_group = _GROUP * _CHUNK
    b_total = b_per_w * _NW

    mesh = plsc.VectorSubcoreMesh(
        core_axis_name="c", subcore_axis_name="s",
        num_cores=_NC, num_subcores=_NS,
    )

    @functools.partial(
        pl.kernel,
        out_type=jax.ShapeDtypeStruct((b_total, d), jnp.float32),
        mesh=mesh,
        scratch_types=[
            pltpu.VMEM((chunks_per_w, _CHUNK), jnp.int32),
            pltpu.VMEM((rows_per_group, d), jnp.float32),
            pltpu.SemaphoreType.DMA,
        ],
        compiler_params=pltpu.CompilerParams(use_tc_tiling_on_sc=False),
    )
    def body(idx_hbm, table_hbm, out_hbm, idx_v, rows_v, gsem):
        wid = lax.axis_index("s") * _NC + lax.axis_index("c")
        base = wid * b_per_w
        pltpu.sync_copy(idx_hbm.at[pl.ds(wid * chunks_per_w, chunks_per_w)],
                        idx_v)

        def group_fn(g, _):
            copies = []
            for j in range(_GROUP):
                copies.append(pltpu.async_copy(
                    table_hbm.at[idx_v.at[g * _GROUP + j]],
                    rows_v.at[pl.ds(j * _CHUNK, _CHUNK)],
                    gsem))
            for c in copies:
                c.wait()
            pltpu.sync_copy(
                rows_v,
                out_hbm.at[pl.ds(base + g * rows_per_group, rows_per_group)])
            return ()

        lax.fori_loop(0, groups, group_fn, (), unroll=False)

    return body(idx2d, table)


def kernel(x, table):
    b_total = x.size
    d = table.shape[1]
    assert b_total % (_NW * _CHUNK * _GROUP) == 0
    b_per_w = b_total // _NW
    idx2d = x.reshape(b_total // _CHUNK, _CHUNK).astype(jnp.int32)
    out = _gather_call(idx2d, table, b_per_w=b_per_w, d=d)
    return out.reshape(x.shape + (d,))


# speedup vs baseline: 1.0133x; 1.0133x over previous
import functools

import jax
import jax.numpy as jnp
from jax import lax
from jax.experimental import pallas as pl
from jax.experimental.pallas import tpu as pltpu
from jax.experimental.pallas import tpu_sc as plsc

_NC = 2
_NS = 16
_NW = _NC * _NS

_CHUNK = 128
_GROUP = 4
_NBUF = 3


@functools.partial(jax.jit, static_argnames=("b_per_w", "d"))
def _gather_call(idx2d, table, *, b_per_w, d):
    chunks_per_w = b_per_w // _CHUNK
    groups = chunks_per_w // _GROUP
    rpg = _GROUP * _CHUNK
    b_total = b_per_w * _NW
    assert groups >= _NBUF + 1

    mesh = plsc.VectorSubcoreMesh(
        core_axis_name="c", subcore_axis_name="s",
        num_cores=_NC, num_subcores=_NS,
    )

    @functools.partial(
        pl.kernel,
        out_type=jax.ShapeDtypeStruct((b_total, d), jnp.float32),
        mesh=mesh,
        scratch_types=[
            pltpu.VMEM((chunks_per_w, _CHUNK), jnp.int32),
            pltpu.VMEM((_NBUF, rpg, d), jnp.float32),
            pltpu.SemaphoreType.DMA((_NBUF,)),
            pltpu.SemaphoreType.DMA((_NBUF,)),
        ],
        compiler_params=pltpu.CompilerParams(use_tc_tiling_on_sc=False),
    )
    def body(idx_hbm, table_hbm, out_hbm, idx_v, rows_v, gsem, wsem):
        wid = lax.axis_index("s") * _NC + lax.axis_index("c")
        base = wid * b_per_w
        pltpu.sync_copy(idx_hbm.at[pl.ds(wid * chunks_per_w, chunks_per_w)],
                        idx_v)

        def fire_g(g, b):
            for j in range(_GROUP):
                pltpu.async_copy(
                    table_hbm.at[idx_v.at[g * _GROUP + j]],
                    rows_v.at[b, pl.ds(j * _CHUNK, _CHUNK)],
                    gsem.at[b])

        def drain_g(b):
            pltpu.make_async_copy(
                table_hbm.at[pl.ds(0, rpg)], rows_v.at[b], gsem.at[b]).wait()

        def fire_w(g, b):
            pltpu.async_copy(rows_v.at[b],
                             out_hbm.at[pl.ds(base + g * rpg, rpg)],
                             wsem.at[b])

        def wait_w(b):
            pltpu.make_async_copy(rows_v.at[b],
                                  out_hbm.at[pl.ds(base, rpg)],
                                  wsem.at[b]).wait()

        fire_g(0, 0)
        fire_g(1, 1)
        drain_g(0)
        fire_w(0, 0)
        fire_g(2, 2)
        drain_g(1)
        fire_w(1, 1)

        def step(g, _):
            b = g % _NBUF
            bn = (g + 1) % _NBUF
            wait_w(bn)
            fire_g(g + 1, bn)
            drain_g(b)
            fire_w(g, b)
            return ()

        lax.fori_loop(2, groups - 1, step, (), unroll=False)

        g = groups - 1
        wait_w((g + 1) % _NBUF)
        drain_g(g % _NBUF)
        fire_w(g, g % _NBUF)
        wait_w((groups - 2) % _NBUF)
        wait_w((groups - 1) % _NBUF)

    return body(idx2d, table)


def kernel(x, table):
    b_total = x.size
    d = table.shape[1]
    assert b_total % (_NW * _CHUNK * _GROUP) == 0
    b_per_w = b_total // _NW
    idx2d = x.reshape(b_total // _CHUNK, _CHUNK).astype(jnp.int32)
    out = _gather_call(idx2d, table, b_per_w=b_per_w, d=d)
    return out.reshape(x.shape + (d,))
